# baseline (speedup 1.0000x reference)
import jax
import jax.numpy as jnp
from jax.experimental import pallas as pl

_CH = 64


def _mm_body(x_ref, w_ref, b_ref, o_ref):
    o_ref[0] = (
        jnp.dot(x_ref[...], w_ref[...], preferred_element_type=jnp.float32)
        + b_ref[0]
    )


def _project(x, W, b):
    N, Din = x.shape
    Dout = W.shape[1]
    H = Dout // 2
    RB = 1000
    nb = N // RB
    b2 = b.reshape(2, 1, H)
    out = pl.pallas_call(
        _mm_body,
        grid=(2, nb),
        in_specs=[
            pl.BlockSpec((RB, Din), lambda c, i: (i, 0)),
            pl.BlockSpec((Din, H), lambda c, i: (0, c)),
            pl.BlockSpec((1, 1, H), lambda c, i: (c, 0, 0)),
        ],
        out_specs=pl.BlockSpec((1, RB, H), lambda c, i: (c, i, 0)),
        out_shape=jax.ShapeDtypeStruct((2, N, H), jnp.float32),
    )(x, W, b2)
    return out.reshape(2 * N, H)


def kernel(x_user, x_item, edge_follows, edge_clicks, edge_rev_clicks,
           W_user, b_user, W_item, b_item):
    N_u = x_user.shape[0]
    N_i = x_item.shape[0]
    H = W_user.shape[1] // 2

    wh_u = _project(x_user, W_user, b_user)
    wh_i = _project(x_item, W_item, b_item)

    def _seg_mean(wh2, src, dst, n_dst):
        m = jnp.take(wh2.reshape(2, -1, H), src, axis=1)
        s = jax.vmap(lambda mm: jax.ops.segment_sum(mm, dst, num_segments=n_dst))(m)
        cnt = jax.ops.segment_sum(jnp.ones((src.shape[0],), jnp.float32), dst,
                                  num_segments=n_dst)
        r = s / jnp.maximum(cnt, 1.0)[None, :, None]
        n_pad = -(-n_dst // _CH) * _CH
        return jnp.pad(r, ((0, 0), (0, n_pad - n_dst), (0, 0))).reshape(-1, H)

    h_f = _seg_mean(wh_u, edge_follows[0], edge_follows[1], N_u)
    h_rev = _seg_mean(wh_i, edge_rev_clicks[0], edge_rev_clicks[1], N_u)
    h_user2 = (h_f + h_rev) * 0.5
    h_item2 = _seg_mean(wh_u, edge_clicks[0], edge_clicks[1], N_i)

    N_pad = -(-N_u // _CH) * _CH
    h_user = (h_user2.reshape(2, N_pad, H)[:, :N_u]
              .transpose(1, 0, 2).reshape(N_u, 2 * H))
    h_item = (h_item2.reshape(2, N_pad, H)[:, :N_i]
              .transpose(1, 0, 2).reshape(N_i, 2 * H))
    return h_user, h_item


# speedup vs baseline: 16.7425x; 16.7425x over previous
import jax
import jax.numpy as jnp
from jax.experimental import pallas as pl


def _mm_body(x_ref, w_ref, b_ref, o_ref):
    o_ref[...] = (
        jnp.dot(x_ref[...], w_ref[...], preferred_element_type=jnp.float32)
        + b_ref[...]
    )


def _project(x, W, b):
    N, Din = x.shape
    Dout = W.shape[1]
    RB = 1000
    nb = N // RB
    b2 = b.reshape(1, Dout)
    return pl.pallas_call(
        _mm_body,
        grid=(nb,),
        in_specs=[
            pl.BlockSpec((RB, Din), lambda i: (i, 0)),
            pl.BlockSpec((Din, Dout), lambda i: (0, 0)),
            pl.BlockSpec((1, Dout), lambda i: (0, 0)),
        ],
        out_specs=pl.BlockSpec((RB, Dout), lambda i: (i, 0)),
        out_shape=jax.ShapeDtypeStruct((N, Dout), jnp.float32),
    )(x, W, b2)


def kernel(x_user, x_item, edge_follows, edge_clicks, edge_rev_clicks,
           W_user, b_user, W_item, b_item):
    N_u = x_user.shape[0]
    N_i = x_item.shape[0]

    wh_u = _project(x_user, W_user, b_user)
    wh_i = _project(x_item, W_item, b_item)

    def _seg_mean(feat, src, dst, n_dst):
        m = jnp.take(feat, src, axis=0)
        s = jax.ops.segment_sum(m, dst, num_segments=n_dst)
        cnt = jax.ops.segment_sum(jnp.ones((src.shape[0],), m.dtype), dst,
                                  num_segments=n_dst)
        return s / jnp.maximum(cnt, 1.0)[:, None]

    h_f = _seg_mean(wh_u, edge_follows[0], edge_follows[1], N_u)
    h_rev = _seg_mean(wh_i, edge_rev_clicks[0], edge_rev_clicks[1], N_u)
    h_user = (h_f + h_rev) * 0.5
    h_item = _seg_mean(wh_u, edge_clicks[0], edge_clicks[1], N_i)
    return h_user, h_item
